# baseline (speedup 1.0000x reference)
import functools

import jax
import jax.numpy as jnp
from jax.experimental import pallas as pl

B, N, L, P, C = 64, 10000, 336, 96, 7
CTX, CAT, GH = 64, 16, 128
COARSE_K, TOPM = 80, 20
ALPHA, TEMP = 0.7, 0.1
D = L * C

N_TILE = 400
N_GRID = N // N_TILE


def _sim_body(q_ref, k_ref, sim_ref):
    q = q_ref[...]
    k = k_ref[...]
    qn = q / (jnp.sqrt(jnp.sum(q * q, axis=1, keepdims=True)) + 1e-8)
    kn = k / (jnp.sqrt(jnp.sum(k * k, axis=1, keepdims=True)) + 1e-8)
    s = jax.lax.dot_general(kn.astype(jnp.bfloat16), qn.astype(jnp.bfloat16),
                            (((1,), (1,)), ((), ())),
                            preferred_element_type=jnp.float32)
    sim_ref[...] = s


@functools.partial(jax.jit, static_argnums=())
def _coarse_sim(qf, kf):
    return pl.pallas_call(
        _sim_body,
        grid=(N_GRID,),
        in_specs=[
            pl.BlockSpec((B, D), lambda i: (0, 0)),
            pl.BlockSpec((N_TILE, D), lambda i: (i, 0)),
        ],
        out_specs=pl.BlockSpec((N_TILE, B), lambda i: (i, 0)),
        out_shape=jax.ShapeDtypeStruct((N, B), jnp.float32),
    )(qf, kf)


def _encode_context(local_state_by_period, dataset_id, sensor_type_id,
                    physical_location_id, hour, day_of_week, month, is_holiday,
                    peak_status_id, emb_dataset, emb_sensor, emb_location,
                    emb_hour, emb_weekday, emb_month, emb_holiday, emb_peak,
                    cat_W1, cat_b1, cat_W2, cat_b2,
                    loc_W1, loc_b1, loc_W2, loc_b2, ln_g, ln_b):
    cat = jnp.concatenate([
        emb_dataset[dataset_id],
        emb_sensor[sensor_type_id],
        emb_location[physical_location_id],
        emb_hour[jnp.clip(hour, 0, 23)],
        emb_weekday[jnp.clip(day_of_week, 0, 6)],
        emb_month[jnp.clip(month, 1, 12)],
        emb_holiday[jnp.clip(is_holiday, 0, 1)],
        emb_peak[jnp.clip(peak_status_id, 0, 1)],
    ], axis=1)
    cat_ctx = jax.nn.gelu(cat @ cat_W1 + cat_b1) @ cat_W2 + cat_b2
    ls = local_state_by_period[:, :3, :]
    loc_ctx = jax.nn.gelu(ls @ loc_W1 + loc_b1) @ loc_W2 + loc_b2
    h = cat_ctx[:, None, :] + loc_ctx
    mu = jnp.mean(h, axis=-1, keepdims=True)
    var = jnp.var(h, axis=-1, keepdims=True)
    h = (h - mu) / jnp.sqrt(var + 1e-5) * ln_g + ln_b
    return h


def kernel(x, pool_x, pool_y, pool_context, local_state_by_period, dataset_id,
           sensor_type_id, physical_location_id, hour, day_of_week, month,
           is_holiday, peak_status_id, emb_dataset, emb_sensor, emb_location,
           emb_hour, emb_weekday, emb_month, emb_holiday, emb_peak,
           cat_W1, cat_b1, cat_W2, cat_b2, loc_W1, loc_b1, loc_W2, loc_b2,
           ln_g, ln_b, gate_W1, gate_b1, gate_W2, gate_b2):
    ctx = _encode_context(local_state_by_period, dataset_id, sensor_type_id,
                          physical_location_id, hour, day_of_week, month,
                          is_holiday, peak_status_id,
                          emb_dataset, emb_sensor, emb_location, emb_hour,
                          emb_weekday, emb_month, emb_holiday, emb_peak,
                          cat_W1, cat_b1, cat_W2, cat_b2,
                          loc_W1, loc_b1, loc_W2, loc_b2, ln_g, ln_b)
    q_ctx = jnp.mean(ctx, axis=1)

    qf = x.reshape(B, D)
    kf = pool_x.reshape(N, D)
    sim = _coarse_sim(qf, kf).T

    coarse_vals, coarse_idx = jax.lax.top_k(sim, COARSE_K)
    cand_ctx = pool_context[coarse_idx]
    qc = q_ctx / (jnp.linalg.norm(q_ctx, axis=-1, keepdims=True) + 1e-8)
    cc = cand_ctx / (jnp.linalg.norm(cand_ctx, axis=-1, keepdims=True) + 1e-8)
    ctx_sim = jnp.sum(qc[:, None, :] * cc, axis=-1)
    gate_in = jnp.concatenate([
        jnp.broadcast_to(q_ctx[:, None, :], cand_ctx.shape),
        cand_ctx,
        coarse_vals[..., None],
        ctx_sim[..., None],
    ], axis=-1)
    gate = jax.nn.gelu(gate_in @ gate_W1 + gate_b1) @ gate_W2 + gate_b2
    score = ALPHA * coarse_vals + (1.0 - ALPHA) * ctx_sim + jnp.squeeze(gate, -1)
    top_vals, top_loc = jax.lax.top_k(score, TOPM)
    topm_idx = jnp.take_along_axis(coarse_idx, top_loc, axis=1)
    w = jax.nn.softmax(top_vals / TEMP, axis=-1)
    y_cand = pool_y[topm_idx]
    out = jnp.sum(w[:, :, None, None] * y_cand, axis=1)
    return out
